# Initial kernel scaffold; baseline (speedup 1.0000x reference)
import jax
import jax.numpy as jnp
from jax.experimental import pallas as pl


def kernel(h, edge_index):
    raise NotImplementedError("write your pallas kernel here")


# speedup vs baseline: 1.0780x; 1.0780x over previous
import functools

import jax
import jax.numpy as jnp
from jax import lax
from jax.experimental import pallas as pl
from jax.experimental.pallas import tpu as pltpu
from jax.experimental.pallas import tpu_sc as plsc

_EPS = 1e-6
_NC = 2
_NS = 16
_NW = _NC * _NS
_L = 16
_D = 128
_E = 320000
_CHUNK = 80


def _sqrt16(x):
    xi = plsc.bitcast(x, jnp.int32)
    yi = jnp.int32(0x5F3759DF) - (xi >> 1)
    y = plsc.bitcast(yi, jnp.float32)
    half_x = 0.5 * x
    for _ in range(3):
        y = y * (1.5 - half_x * y * y)
    return x * y


def _make_kernel():
    mesh = plsc.VectorSubcoreMesh(
        core_axis_name="c", subcore_axis_name="s",
        num_cores=_NC, num_subcores=_NS,
    )
    epw = _E // _NW
    nchunks = epw // _CHUNK

    @functools.partial(
        pl.kernel,
        out_type=jax.ShapeDtypeStruct((_E,), jnp.float32),
        mesh=mesh,
        compiler_params=pltpu.CompilerParams(needs_layout_passes=False),
        scratch_types=[
            pltpu.VMEM((_CHUNK,), jnp.int32),
            pltpu.VMEM((_CHUNK,), jnp.int32),
            pltpu.VMEM((_CHUNK, _D), jnp.float32),
            pltpu.VMEM((_CHUNK, _D), jnp.float32),
            pltpu.VMEM((_CHUNK,), jnp.float32),
            pltpu.SemaphoreType.DMA,
            pltpu.SemaphoreType.DMA,
        ],
    )
    def ep_kernel(h_hbm, src_hbm, dst_hbm, out_hbm,
                  sidx, didx, srows, drows, oscore, sem_s, sem_d):
        wid = lax.axis_index("s") * _NC + lax.axis_index("c")
        base = wid * epw
        lane = lax.iota(jnp.int32, _L)

        def chunk_body(c, carry):
            off = base + c * _CHUNK
            pltpu.sync_copy(src_hbm.at[pl.ds(off, _CHUNK)], sidx)
            pltpu.sync_copy(dst_hbm.at[pl.ds(off, _CHUNK)], didx)
            cp_s = pltpu.async_copy(h_hbm.at[sidx], srows, sem_s)
            cp_d = pltpu.async_copy(h_hbm.at[didx], drows, sem_d)
            cp_s.wait()
            cp_d.wait()
            for g in range(_CHUNK // _L):
                rows = lane + (g * _L)

                def feat_body(fo, acc):
                    for fi in range(16):
                        cols = jnp.full((_L,), fo * 16 + fi, jnp.int32)
                        s = plsc.load_gather(srows, [rows, cols])
                        d = plsc.load_gather(drows, [rows, cols])
                        t = s - d + _EPS
                        acc = acc + t * t
                    return acc

                acc = lax.fori_loop(0, _D // 16, feat_body,
                                    jnp.zeros((_L,), jnp.float32))
                oscore[pl.ds(g * _L, _L)] = _sqrt16(acc)
            pltpu.sync_copy(oscore, out_hbm.at[pl.ds(off, _CHUNK)])
            return carry

        lax.fori_loop(0, nchunks, chunk_body, 0)

    return ep_kernel


_EP_KERNEL = _make_kernel()


def kernel(h, edge_index):
    ei = edge_index.astype(jnp.int32)
    return _EP_KERNEL(h, ei[0], ei[1])
